# baseline (speedup 1.0000x reference)
import functools

import jax
import jax.numpy as jnp
from jax import lax
from jax.experimental import pallas as pl
from jax.experimental.pallas import tpu as pltpu
from jax.experimental.pallas import tpu_sc as plsc

NX = 100000
NQ = 4096
D = 64
C = 64
K = 6
KPAD = 8

BQ = 512
BN = 8192
NB = 13
NPAD = NB * BN

NW = 32
QPW = NQ // NW
PPW = QPW * K
NCH = PPW // 128


def _topk_body(q_ref, x_ref, xsq_ref, val_ref, idx_ref, cval_ref, cidx_ref):
    j = pl.program_id(1)
    inf = jnp.float32(jnp.inf)
    imax = jnp.int32(2**31 - 1)

    @pl.when(j == 0)
    def _init():
        cval_ref[...] = jnp.full((NB, BQ, KPAD), jnp.inf, jnp.float32)
        cidx_ref[...] = jnp.zeros((NB, BQ, KPAD), jnp.int32)

    qb = q_ref[...]
    xb = x_ref[...]
    s = -2.0 * lax.dot_general(qb, xb, (((1,), (1,)), ((), ())),
                               preferred_element_type=jnp.float32)
    s = s + xsq_ref[...]

    liota = lax.broadcasted_iota(jnp.int32, (BQ, BN), 1)

    bvals, bidxs = [], []
    for _ in range(K):
        m = jnp.min(s, axis=1, keepdims=True)
        midx = jnp.where(s == m, liota, imax)
        am = jnp.min(midx, axis=1, keepdims=True)
        bvals.append(m)
        bidxs.append(j * BN + am)
        s = jnp.where(midx == am, inf, s)

    pad_v = jnp.full((BQ, KPAD - K), jnp.inf, jnp.float32)
    pad_i = jnp.zeros((BQ, KPAD - K), jnp.int32)
    cval_ref[j] = jnp.concatenate(bvals + [pad_v], 1)
    cidx_ref[j] = jnp.concatenate(bidxs + [pad_i], 1)

    @pl.when(j == NB - 1)
    def _finish():
        cval = cval_ref[...]
        cidx = cidx_ref[...]
        nv, ni = [], []
        for _ in range(K):
            m = jnp.min(jnp.min(cval, axis=0), axis=1, keepdims=True)
            eq = cval == m[None, :, :]
            ci = jnp.min(jnp.min(jnp.where(eq, cidx, imax), axis=0), axis=1,
                         keepdims=True)
            nv.append(m)
            ni.append(ci)
            cval = jnp.where(eq & (cidx == ci[None, :, :]), inf, cval)

        qsq = jnp.sum(qb * qb, axis=1, keepdims=True)
        vals = jnp.concatenate(nv + [pad_v], axis=1)
        val_ref[...] = jnp.sqrt(jnp.maximum(vals + qsq, 0.0))
        idx_ref[...] = jnp.concatenate(ni + [pad_i], axis=1)


_topk = pl.pallas_call(
    _topk_body,
    grid=(NQ // BQ, NB),
    in_specs=[
        pl.BlockSpec((BQ, D), lambda i, j: (i, 0)),
        pl.BlockSpec((BN, D), lambda i, j: (j, 0)),
        pl.BlockSpec((1, BN), lambda i, j: (0, j)),
    ],
    out_specs=[
        pl.BlockSpec((BQ, KPAD), lambda i, j: (i, 0)),
        pl.BlockSpec((BQ, KPAD), lambda i, j: (i, 0)),
    ],
    out_shape=[
        jax.ShapeDtypeStruct((NQ, KPAD), jnp.float32),
        jax.ShapeDtypeStruct((NQ, KPAD), jnp.int32),
    ],
    scratch_shapes=[
        pltpu.VMEM((NB, BQ, KPAD), jnp.float32),
        pltpu.VMEM((NB, BQ, KPAD), jnp.int32),
    ],
    compiler_params=pltpu.CompilerParams(
        dimension_semantics=("parallel", "arbitrary"),
        vmem_limit_bytes=100 * 1024 * 1024,
    ),
)


def _combine_body(zt_hbm, ix_hbm, w_hbm, out_hbm, idx_v, w_v, rows_v, out_v,
                  sem):
    wid = lax.axis_index("s") * 2 + lax.axis_index("c")

    pltpu.sync_copy(ix_hbm.at[wid], idx_v)
    pltpu.sync_copy(w_hbm.at[wid], w_v)

    copies = []
    for ch in range(NCH):
        copies.append(
            pltpu.async_copy(zt_hbm.at[idx_v.at[ch]],
                             rows_v.at[pl.ds(ch * 128, 128)], sem))
    for cp in copies:
        cp.wait()

    def qbody(qi, carry):
        ws = [w_v[qi * K + k, :] for k in range(K)]
        for c4 in range(C // 16):
            sl = pl.ds(c4 * 16, 16)
            acc = ws[0] * rows_v[qi * K, sl]
            for k in range(1, K):
                acc = acc + ws[k] * rows_v[qi * K + k, sl]
            out_v[qi, sl] = acc
        return carry

    lax.fori_loop(0, QPW, qbody, 0)
    pltpu.sync_copy(out_v, out_hbm.at[pl.ds(wid * QPW, QPW)])


@functools.lru_cache(maxsize=1)
def _get_combine():
    return functools.partial(
        pl.kernel,
        out_type=jax.ShapeDtypeStruct((NQ, C), jnp.float32),
        mesh=plsc.VectorSubcoreMesh(core_axis_name="c", subcore_axis_name="s"),
        scratch_types=[
            pltpu.VMEM((NCH, 128), jnp.int32),
            pltpu.VMEM((PPW, 16), jnp.float32),
            pltpu.VMEM((PPW, C), jnp.float32),
            pltpu.VMEM((QPW, C), jnp.float32),
            pltpu.SemaphoreType.DMA,
        ],
        compiler_params=pltpu.CompilerParams(use_tc_tiling_on_sc=False),
    )(_combine_body)


@jax.jit
def kernel(x, q, z):
    xpad = jnp.pad(x, ((0, NPAD - NX), (0, 0)))
    xsq = jnp.sum(xpad * xpad, axis=1)
    xsq = jnp.where(jnp.arange(NPAD) < NX, xsq, 1e30).reshape(1, NPAD)

    dist, ix = _topk(q, xpad, xsq)
    dist = dist[:, :K]
    ix = ix[:, :K]

    sigma_squared = jnp.square(jnp.median(dist)) / 9.0
    w = (1.0 / jnp.sqrt(2.0 * jnp.pi * sigma_squared)) * jnp.exp(
        -0.5 * dist * dist / sigma_squared)
    w = w / jnp.sum(w, axis=-1, keepdims=True)
    w = jnp.nan_to_num(w, nan=1.0 / K).astype(jnp.float32)

    zt = z.T
    ixr = ix.reshape(NW, NCH, 128)
    wb = jnp.broadcast_to(w.reshape(NW, PPW, 1), (NW, PPW, 16))
    out = _get_combine()(zt, ixr, wb)
    return out.T


# speedup vs baseline: 1.5221x; 1.5221x over previous
import functools

import jax
import jax.numpy as jnp
from jax import lax
from jax.experimental import pallas as pl
from jax.experimental.pallas import tpu as pltpu
from jax.experimental.pallas import tpu_sc as plsc

NX = 100000
NQ = 4096
D = 64
C = 64
K = 6
KPAD = 8

BQ = 512
BN = 5120
NB = 20
NPAD = NB * BN

NW = 32
QPW = NQ // NW
PPW = QPW * K
NCH = PPW // 128


def _topk_body(q_ref, x_ref, xsq_ref, val_ref, idx_ref, cval_ref, cidx_ref):
    j = pl.program_id(1)
    inf = jnp.float32(jnp.inf)
    imax = jnp.int32(2**31 - 1)

    @pl.when(j == 0)
    def _init():
        cval_ref[...] = jnp.full((NB, BQ, KPAD), jnp.inf, jnp.float32)
        cidx_ref[...] = jnp.zeros((NB, BQ, KPAD), jnp.int32)

    qb = q_ref[...]
    xb = x_ref[...]
    s = -2.0 * lax.dot_general(qb, xb, (((1,), (1,)), ((), ())),
                               preferred_element_type=jnp.float32)
    s = s + xsq_ref[...]

    liota = lax.broadcasted_iota(jnp.int32, (BQ, BN), 1)

    bvals, bidxs = [], []
    for _ in range(K):
        m = jnp.min(s, axis=1, keepdims=True)
        midx = jnp.where(s == m, liota, imax)
        am = jnp.min(midx, axis=1, keepdims=True)
        bvals.append(m)
        bidxs.append(j * BN + am)
        s = jnp.where(midx == am, inf, s)

    pad_v = jnp.full((BQ, KPAD - K), jnp.inf, jnp.float32)
    pad_i = jnp.zeros((BQ, KPAD - K), jnp.int32)
    cval_ref[j] = jnp.concatenate(bvals + [pad_v], 1)
    cidx_ref[j] = jnp.concatenate(bidxs + [pad_i], 1)

    @pl.when(j == NB - 1)
    def _finish():
        cval = cval_ref[...]
        cidx = cidx_ref[...]
        nv, ni = [], []
        for _ in range(K):
            m = jnp.min(jnp.min(cval, axis=0), axis=1, keepdims=True)
            eq = cval == m[None, :, :]
            ci = jnp.min(jnp.min(jnp.where(eq, cidx, imax), axis=0), axis=1,
                         keepdims=True)
            nv.append(m)
            ni.append(ci)
            cval = jnp.where(eq & (cidx == ci[None, :, :]), inf, cval)

        qsq = jnp.sum(qb * qb, axis=1, keepdims=True)
        vals = jnp.concatenate(nv + [pad_v], axis=1)
        val_ref[...] = jnp.sqrt(jnp.maximum(vals + qsq, 0.0))
        idx_ref[...] = jnp.concatenate(ni + [pad_i], axis=1)


_topk = pl.pallas_call(
    _topk_body,
    grid=(NQ // BQ, NB),
    in_specs=[
        pl.BlockSpec((BQ, D), lambda i, j: (i, 0)),
        pl.BlockSpec((BN, D), lambda i, j: (j, 0)),
        pl.BlockSpec((1, BN), lambda i, j: (0, j)),
    ],
    out_specs=[
        pl.BlockSpec((BQ, KPAD), lambda i, j: (i, 0)),
        pl.BlockSpec((BQ, KPAD), lambda i, j: (i, 0)),
    ],
    out_shape=[
        jax.ShapeDtypeStruct((NQ, KPAD), jnp.float32),
        jax.ShapeDtypeStruct((NQ, KPAD), jnp.int32),
    ],
    scratch_shapes=[
        pltpu.VMEM((NB, BQ, KPAD), jnp.float32),
        pltpu.VMEM((NB, BQ, KPAD), jnp.int32),
    ],
    compiler_params=pltpu.CompilerParams(
        dimension_semantics=("parallel", "arbitrary"),
        vmem_limit_bytes=100 * 1024 * 1024,
    ),
)


def _combine_body(zt_hbm, ix_hbm, w_hbm, out_hbm, idx_v, w_v, rows_v, out_v,
                  sem):
    wid = lax.axis_index("s") * 2 + lax.axis_index("c")

    pltpu.sync_copy(ix_hbm.at[wid], idx_v)
    pltpu.sync_copy(w_hbm.at[wid], w_v)

    copies = []
    for ch in range(NCH):
        copies.append(
            pltpu.async_copy(zt_hbm.at[idx_v.at[ch]],
                             rows_v.at[pl.ds(ch * 128, 128)], sem))
    for cp in copies:
        cp.wait()

    def qbody(qi, carry):
        ws = [w_v[qi * K + k, :] for k in range(K)]
        for c4 in range(C // 16):
            sl = pl.ds(c4 * 16, 16)
            acc = ws[0] * rows_v[qi * K, sl]
            for k in range(1, K):
                acc = acc + ws[k] * rows_v[qi * K + k, sl]
            out_v[qi, sl] = acc
        return carry

    lax.fori_loop(0, QPW, qbody, 0)
    pltpu.sync_copy(out_v, out_hbm.at[pl.ds(wid * QPW, QPW)])


@functools.lru_cache(maxsize=1)
def _get_combine():
    return functools.partial(
        pl.kernel,
        out_type=jax.ShapeDtypeStruct((NQ, C), jnp.float32),
        mesh=plsc.VectorSubcoreMesh(core_axis_name="c", subcore_axis_name="s"),
        scratch_types=[
            pltpu.VMEM((NCH, 128), jnp.int32),
            pltpu.VMEM((PPW, 16), jnp.float32),
            pltpu.VMEM((PPW, C), jnp.float32),
            pltpu.VMEM((QPW, C), jnp.float32),
            pltpu.SemaphoreType.DMA,
        ],
        compiler_params=pltpu.CompilerParams(use_tc_tiling_on_sc=False),
    )(_combine_body)


@jax.jit
def kernel(x, q, z):
    xpad = jnp.pad(x, ((0, NPAD - NX), (0, 0)))
    xsq = jnp.sum(xpad * xpad, axis=1)
    xsq = jnp.where(jnp.arange(NPAD) < NX, xsq, 1e30).reshape(1, NPAD)

    dist, ix = _topk(q, xpad, xsq)
    dist = dist[:, :K]
    ix = ix[:, :K]

    sigma_squared = jnp.square(jnp.median(dist)) / 9.0
    w = (1.0 / jnp.sqrt(2.0 * jnp.pi * sigma_squared)) * jnp.exp(
        -0.5 * dist * dist / sigma_squared)
    w = w / jnp.sum(w, axis=-1, keepdims=True)
    w = jnp.nan_to_num(w, nan=1.0 / K).astype(jnp.float32)

    zt = z.T
    ixr = ix.reshape(NW, NCH, 128)
    wb = jnp.broadcast_to(w.reshape(NW, PPW, 1), (NW, PPW, 16))
    out = _get_combine()(zt, ixr, wb)
    return out.T


# speedup vs baseline: 1.5490x; 1.0177x over previous
import functools

import jax
import jax.numpy as jnp
from jax import lax
from jax.experimental import pallas as pl
from jax.experimental.pallas import tpu as pltpu
from jax.experimental.pallas import tpu_sc as plsc

NX = 100000
NQ = 4096
D = 64
C = 64
K = 6
KPAD = 8

BQ = 512
BN = 6400
NB = 16
NPAD = NB * BN

NW = 32
QPW = NQ // NW
PPW = QPW * K
NCH = PPW // 128


def _topk_body(q_ref, x_ref, xsq_ref, val_ref, idx_ref, cval_ref, cidx_ref):
    j = pl.program_id(1)
    inf = jnp.float32(jnp.inf)
    imax = jnp.int32(2**31 - 1)

    @pl.when(j == 0)
    def _init():
        cval_ref[...] = jnp.full((NB, BQ, KPAD), jnp.inf, jnp.float32)
        cidx_ref[...] = jnp.zeros((NB, BQ, KPAD), jnp.int32)

    qb = q_ref[...]
    xb = x_ref[...]
    s = -2.0 * lax.dot_general(qb, xb, (((1,), (1,)), ((), ())),
                               preferred_element_type=jnp.float32)
    s = s + xsq_ref[...]

    liota = lax.broadcasted_iota(jnp.int32, (BQ, BN), 1)

    bvals, bidxs = [], []
    for _ in range(K):
        m = jnp.min(s, axis=1, keepdims=True)
        midx = jnp.where(s == m, liota, imax)
        am = jnp.min(midx, axis=1, keepdims=True)
        bvals.append(m)
        bidxs.append(j * BN + am)
        s = jnp.where(midx == am, inf, s)

    pad_v = jnp.full((BQ, KPAD - K), jnp.inf, jnp.float32)
    pad_i = jnp.zeros((BQ, KPAD - K), jnp.int32)
    cval_ref[j] = jnp.concatenate(bvals + [pad_v], 1)
    cidx_ref[j] = jnp.concatenate(bidxs + [pad_i], 1)

    @pl.when(j == NB - 1)
    def _finish():
        cval = cval_ref[...]
        cidx = cidx_ref[...]
        nv, ni = [], []
        for _ in range(K):
            m = jnp.min(jnp.min(cval, axis=0), axis=1, keepdims=True)
            eq = cval == m[None, :, :]
            ci = jnp.min(jnp.min(jnp.where(eq, cidx, imax), axis=0), axis=1,
                         keepdims=True)
            nv.append(m)
            ni.append(ci)
            cval = jnp.where(eq & (cidx == ci[None, :, :]), inf, cval)

        qsq = jnp.sum(qb * qb, axis=1, keepdims=True)
        vals = jnp.concatenate(nv + [pad_v], axis=1)
        val_ref[...] = jnp.sqrt(jnp.maximum(vals + qsq, 0.0))
        idx_ref[...] = jnp.concatenate(ni + [pad_i], axis=1)


_topk = pl.pallas_call(
    _topk_body,
    grid=(NQ // BQ, NB),
    in_specs=[
        pl.BlockSpec((BQ, D), lambda i, j: (i, 0)),
        pl.BlockSpec((BN, D), lambda i, j: (j, 0)),
        pl.BlockSpec((1, BN), lambda i, j: (0, j)),
    ],
    out_specs=[
        pl.BlockSpec((BQ, KPAD), lambda i, j: (i, 0)),
        pl.BlockSpec((BQ, KPAD), lambda i, j: (i, 0)),
    ],
    out_shape=[
        jax.ShapeDtypeStruct((NQ, KPAD), jnp.float32),
        jax.ShapeDtypeStruct((NQ, KPAD), jnp.int32),
    ],
    scratch_shapes=[
        pltpu.VMEM((NB, BQ, KPAD), jnp.float32),
        pltpu.VMEM((NB, BQ, KPAD), jnp.int32),
    ],
    compiler_params=pltpu.CompilerParams(
        dimension_semantics=("parallel", "arbitrary"),
        vmem_limit_bytes=100 * 1024 * 1024,
    ),
)


def _combine_body(zt_hbm, ix_hbm, w_hbm, out_hbm, idx_v, w_v, rows_v, out_v,
                  sem):
    wid = lax.axis_index("s") * 2 + lax.axis_index("c")

    pltpu.sync_copy(ix_hbm.at[wid], idx_v)
    pltpu.sync_copy(w_hbm.at[wid], w_v)

    copies = []
    for ch in range(NCH):
        copies.append(
            pltpu.async_copy(zt_hbm.at[idx_v.at[ch]],
                             rows_v.at[pl.ds(ch * 128, 128)], sem))
    for cp in copies:
        cp.wait()

    def qbody(qi, carry):
        ws = [w_v[qi * K + k, :] for k in range(K)]
        for c4 in range(C // 16):
            sl = pl.ds(c4 * 16, 16)
            acc = ws[0] * rows_v[qi * K, sl]
            for k in range(1, K):
                acc = acc + ws[k] * rows_v[qi * K + k, sl]
            out_v[qi, sl] = acc
        return carry

    lax.fori_loop(0, QPW, qbody, 0)
    pltpu.sync_copy(out_v, out_hbm.at[pl.ds(wid * QPW, QPW)])


@functools.lru_cache(maxsize=1)
def _get_combine():
    return functools.partial(
        pl.kernel,
        out_type=jax.ShapeDtypeStruct((NQ, C), jnp.float32),
        mesh=plsc.VectorSubcoreMesh(core_axis_name="c", subcore_axis_name="s"),
        scratch_types=[
            pltpu.VMEM((NCH, 128), jnp.int32),
            pltpu.VMEM((PPW, 16), jnp.float32),
            pltpu.VMEM((PPW, C), jnp.float32),
            pltpu.VMEM((QPW, C), jnp.float32),
            pltpu.SemaphoreType.DMA,
        ],
        compiler_params=pltpu.CompilerParams(use_tc_tiling_on_sc=False),
    )(_combine_body)


@jax.jit
def kernel(x, q, z):
    xpad = jnp.pad(x, ((0, NPAD - NX), (0, 0)))
    xsq = jnp.sum(xpad * xpad, axis=1)
    xsq = jnp.where(jnp.arange(NPAD) < NX, xsq, 1e30).reshape(1, NPAD)

    dist, ix = _topk(q, xpad, xsq)
    dist = dist[:, :K]
    ix = ix[:, :K]

    sigma_squared = jnp.square(jnp.median(dist)) / 9.0
    w = (1.0 / jnp.sqrt(2.0 * jnp.pi * sigma_squared)) * jnp.exp(
        -0.5 * dist * dist / sigma_squared)
    w = w / jnp.sum(w, axis=-1, keepdims=True)
    w = jnp.nan_to_num(w, nan=1.0 / K).astype(jnp.float32)

    zt = z.T
    ixr = ix.reshape(NW, NCH, 128)
    wb = jnp.broadcast_to(w.reshape(NW, PPW, 1), (NW, PPW, 16))
    out = _get_combine()(zt, ixr, wb)
    return out.T
